# baseline (speedup 1.0000x reference)
import functools

import jax
import jax.numpy as jnp
from jax import lax
from jax.experimental import pallas as pl
from jax.experimental.pallas import tpu as pltpu
from jax.experimental.pallas import tpu_sc as plsc

_SIZE_IN = 17
_COLS = [2 ** i for i in range(_SIZE_IN)]
_B = 1024
_W = 100000
_NC = 2
_NS = 16
_NW = _NC * _NS
_ROWS = _B // _NW
_L = 16


_N_ELEMS = _SIZE_IN * _ROWS


def _sc_body(flat_hbm, out_hbm, idx_v, vals_v, sem):
    wid = lax.axis_index("s") * _NC + lax.axis_index("c")
    base_row = wid * _ROWS
    r16 = lax.iota(jnp.int32, 16)
    for i in range(_SIZE_IN):
        for h in range(_ROWS // _L):
            rows = base_row + h * _L + r16
            idx_v[pl.ds((i * (_ROWS // _L) + h) * _L, _L)] = (
                rows * _W + _COLS[i]
            )
    pltpu.async_copy(flat_hbm.at[idx_v], vals_v, sem).wait()
    for k in range(_N_ELEMS // _L):
        v = vals_v[pl.ds(k * _L, _L)]
        vals_v[pl.ds(k * _L, _L)] = v * 12.0 - 6.0
    pltpu.sync_copy(vals_v, out_hbm.at[wid])


def kernel(input_var):
    flat = input_var.reshape(-1)
    mesh = plsc.VectorSubcoreMesh(core_axis_name="c", subcore_axis_name="s")
    out = pl.kernel(
        _sc_body,
        out_type=jax.ShapeDtypeStruct((_NW, _N_ELEMS), jnp.float32),
        mesh=mesh,
        scratch_types=[
            pltpu.VMEM((_N_ELEMS,), jnp.int32),
            pltpu.VMEM((_N_ELEMS,), jnp.float32),
            pltpu.SemaphoreType.DMA,
        ],
    )(flat)
    return (
        out.reshape(_NW, _SIZE_IN, _ROWS)
        .transpose(0, 2, 1)
        .reshape(_B, _SIZE_IN)
    )
